# Initial kernel scaffold; baseline (speedup 1.0000x reference)
import jax
import jax.numpy as jnp
from jax.experimental import pallas as pl


def kernel(dense, cat_idx, seq_idx, cat_tables, seq_tables, W1, b1, W2, b2, Wf, bf):
    raise NotImplementedError("write your pallas kernel here")


# speedup vs baseline: 19.8281x; 19.8281x over previous
import functools

import jax
import jax.numpy as jnp
from jax import lax
from jax.experimental import pallas as pl
from jax.experimental.pallas import tpu as pltpu
from jax.experimental.pallas import tpu_sc as plsc

B = 16384
NUM_DENSE = 13
NUM_CAT = 26
NUM_SEQ = 2
VOCAB = 100000
EMB = 16
SEQ_LEN = 50
HID1 = 96
HID2 = 32
CAT_COLS = NUM_CAT * EMB
EMB_COLS = CAT_COLS + NUM_SEQ * EMB

NW = 32
ROWS_W = B // NW
CAT_SUB = 128
CAT_CHUNK = CAT_SUB * NUM_CAT
SEQ_SUB = 64
SEQ_CHUNK = SEQ_SUB * SEQ_LEN

BLK = 512
NBLK = B // BLK


@functools.cache
def _make_sc_gather():
    mesh = plsc.VectorSubcoreMesh(core_axis_name="c", subcore_axis_name="s")
    return functools.partial(
        pl.kernel,
        out_type=(
            jax.ShapeDtypeStruct((B * NUM_CAT, EMB), jnp.float32),
            jax.ShapeDtypeStruct((B, NUM_SEQ * EMB), jnp.float32),
        ),
        mesh=mesh,
        scratch_types=[
            pltpu.VMEM((CAT_CHUNK,), jnp.int32),
            pltpu.VMEM((CAT_CHUNK, EMB), jnp.float32),
            pltpu.VMEM((SEQ_CHUNK,), jnp.int32),
            pltpu.VMEM((SEQ_CHUNK, EMB), jnp.float32),
            pltpu.VMEM((SEQ_SUB, NUM_SEQ * EMB), jnp.float32),
            pltpu.SemaphoreType.DMA,
        ],
        compiler_params=pltpu.CompilerParams(use_tc_tiling_on_sc=False),
    )(_sc_gather_body)


def _sc_gather_body(cat_idx_hbm, seq_idx_hbm, cat_tab_hbm, seq_tab_hbm,
                    cat_out_hbm, seq_out_hbm,
                    cidx_v, crows_v, sidx_v, srows_v, pool_v, sem):
    wid = lax.axis_index("s") * 2 + lax.axis_index("c")
    base = wid * ROWS_W

    for k in range(ROWS_W // CAT_SUB):
        off = (base + k * CAT_SUB) * NUM_CAT
        pltpu.sync_copy(cat_idx_hbm.at[pl.ds(off, CAT_CHUNK)], cidx_v)
        pltpu.async_copy(cat_tab_hbm.at[cidx_v], crows_v, sem).wait()
        pltpu.sync_copy(crows_v, cat_out_hbm.at[pl.ds(off, CAT_CHUNK), :])

    for j in range(ROWS_W // SEQ_SUB):
        sub = base + j * SEQ_SUB
        for s in range(NUM_SEQ):
            off = (s * B + sub) * SEQ_LEN
            pltpu.sync_copy(seq_idx_hbm.at[pl.ds(off, SEQ_CHUNK)], sidx_v)
            pltpu.async_copy(seq_tab_hbm.at[sidx_v], srows_v, sem).wait()

            def _pool_row(i, _):
                a0 = srows_v[i * SEQ_LEN, :]
                a1 = srows_v[i * SEQ_LEN + 1, :]
                a2 = srows_v[i * SEQ_LEN + 2, :]
                a3 = srows_v[i * SEQ_LEN + 3, :]
                for l in range(4, SEQ_LEN, 4):
                    a0 = a0 + srows_v[i * SEQ_LEN + l, :]
                    a1 = a1 + srows_v[i * SEQ_LEN + l + 1, :]
                    a2 = a2 + srows_v[i * SEQ_LEN + l + 2, :]
                    a3 = a3 + srows_v[i * SEQ_LEN + l + 3, :]
                pool_v[i, pl.ds(s * EMB, EMB)] = (a0 + a1) + (a2 + a3)
                return 0

            lax.fori_loop(0, SEQ_SUB, _pool_row, 0)
        pltpu.sync_copy(pool_v, seq_out_hbm.at[pl.ds(sub, SEQ_SUB), :])


def _mlp1_body(dense_ref, cat_ref, seq_ref, w1d_ref, w1c_ref, w1s_ref, b1_ref,
               wfw_ref, wfx_ref, bf_ref,
               x1_ref, plog_ref, s1_ref, q1_ref):
    i = pl.program_id(0)
    dense = dense_ref[...]
    cat = cat_ref[...]
    seq = seq_ref[...]
    x1 = (jnp.dot(dense, w1d_ref[...], preferred_element_type=jnp.float32)
          + jnp.dot(cat, w1c_ref[...], preferred_element_type=jnp.float32)
          + jnp.dot(seq, w1s_ref[...], preferred_element_type=jnp.float32)
          + b1_ref[...])
    x1_ref[...] = x1

    cross = jnp.concatenate(
        [cat[:, 0:16] * cat[:, 16:32], cat[:, 32:48] * cat[:, 48:64],
         cat[:, 64:80] * cat[:, 80:96], cat[:, 96:112] * cat[:, 112:128]],
        axis=1)
    plog = (jnp.sum(dense * wfw_ref[:, 0:NUM_DENSE], axis=1, keepdims=True)
            + jnp.sum(cat * wfw_ref[:, NUM_DENSE:], axis=1, keepdims=True)
            + jnp.sum(cross * wfx_ref[...], axis=1, keepdims=True)
            + bf_ref[...])
    plog_ref[...] = plog

    @pl.when(i == 0)
    def _():
        s1_ref[...] = jnp.zeros_like(s1_ref)
        q1_ref[...] = jnp.zeros_like(q1_ref)
    s1_ref[...] += jnp.sum(x1, axis=0, keepdims=True)
    q1_ref[...] += jnp.sum(x1 * x1, axis=0, keepdims=True)


def _mlp2_body(x1_ref, s1_ref, q1_ref, w2_ref, b2_ref,
               x2_ref, s2_ref, q2_ref):
    i = pl.program_id(0)
    m = s1_ref[...] / B
    v = q1_ref[...] / B - m * m
    inv = lax.rsqrt(v + 1e-5)
    h1 = jnp.maximum((x1_ref[...] - m) * inv, 0.0)
    x2 = jnp.dot(h1, w2_ref[...], preferred_element_type=jnp.float32) + b2_ref[...]
    x2_ref[...] = x2

    @pl.when(i == 0)
    def _():
        s2_ref[...] = jnp.zeros_like(s2_ref)
        q2_ref[...] = jnp.zeros_like(q2_ref)
    s2_ref[...] += jnp.sum(x2, axis=0, keepdims=True)
    q2_ref[...] += jnp.sum(x2 * x2, axis=0, keepdims=True)


def _mlp3_body(x2_ref, s2_ref, q2_ref, plog_ref, wfh_ref, out_ref):
    m = s2_ref[...] / B
    v = q2_ref[...] / B - m * m
    inv = lax.rsqrt(v + 1e-5)
    h = jnp.maximum((x2_ref[...] - m) * inv, 0.0)
    logit = plog_ref[...] + jnp.sum(h * wfh_ref[...], axis=1, keepdims=True)
    out_ref[...] = jax.nn.sigmoid(logit)


def _const_spec(shape):
    return pl.BlockSpec(shape, lambda i: (0,) * len(shape))


def _batch_spec(cols):
    return pl.BlockSpec((BLK, cols), lambda i: (i, 0))


def _mlp(dense, cat, seq, W1, b1, W2, b2, Wf, bf):
    w1d = W1[:NUM_DENSE]
    w1c = W1[NUM_DENSE:NUM_DENSE + CAT_COLS]
    w1s = W1[NUM_DENSE + CAT_COLS:]
    b1r = b1.reshape(1, HID1)
    b2r = b2.reshape(1, HID2)
    wfw = Wf[: NUM_DENSE + CAT_COLS, 0].reshape(1, -1)
    wfx = Wf[NUM_DENSE + CAT_COLS: NUM_DENSE + CAT_COLS + 4 * EMB, 0].reshape(1, -1)
    wfh = Wf[NUM_DENSE + CAT_COLS + 4 * EMB:, 0].reshape(1, HID2)
    bfr = bf.reshape(1, 1)

    x1, plog, s1, q1 = pl.pallas_call(
        _mlp1_body,
        grid=(NBLK,),
        in_specs=[
            _batch_spec(NUM_DENSE), _batch_spec(CAT_COLS),
            _batch_spec(NUM_SEQ * EMB),
            _const_spec((NUM_DENSE, HID1)), _const_spec((CAT_COLS, HID1)),
            _const_spec((NUM_SEQ * EMB, HID1)),
            _const_spec((1, HID1)), _const_spec((1, NUM_DENSE + CAT_COLS)),
            _const_spec((1, 4 * EMB)), _const_spec((1, 1)),
        ],
        out_specs=[
            _batch_spec(HID1), _batch_spec(1),
            _const_spec((1, HID1)), _const_spec((1, HID1)),
        ],
        out_shape=[
            jax.ShapeDtypeStruct((B, HID1), jnp.float32),
            jax.ShapeDtypeStruct((B, 1), jnp.float32),
            jax.ShapeDtypeStruct((1, HID1), jnp.float32),
            jax.ShapeDtypeStruct((1, HID1), jnp.float32),
        ],
    )(dense, cat, seq, w1d, w1c, w1s, b1r, wfw, wfx, bfr)

    x2, s2, q2 = pl.pallas_call(
        _mlp2_body,
        grid=(NBLK,),
        in_specs=[
            _batch_spec(HID1), _const_spec((1, HID1)), _const_spec((1, HID1)),
            _const_spec((HID1, HID2)), _const_spec((1, HID2)),
        ],
        out_specs=[
            _batch_spec(HID2), _const_spec((1, HID2)), _const_spec((1, HID2)),
        ],
        out_shape=[
            jax.ShapeDtypeStruct((B, HID2), jnp.float32),
            jax.ShapeDtypeStruct((1, HID2), jnp.float32),
            jax.ShapeDtypeStruct((1, HID2), jnp.float32),
        ],
    )(x1, s1, q1, W2, b2r)

    out = pl.pallas_call(
        _mlp3_body,
        grid=(NBLK,),
        in_specs=[
            _batch_spec(HID2), _const_spec((1, HID2)), _const_spec((1, HID2)),
            _batch_spec(1), _const_spec((1, HID2)),
        ],
        out_specs=_batch_spec(1),
        out_shape=jax.ShapeDtypeStruct((B, 1), jnp.float32),
    )(x2, s2, q2, plog, wfh)
    return out


def kernel(dense, cat_idx, seq_idx, cat_tables, seq_tables,
           W1, b1, W2, b2, Wf, bf):
    cat_idx_off = (cat_idx
                   + jnp.arange(NUM_CAT, dtype=jnp.int32) * VOCAB).reshape(-1)
    seq_idx_off = (seq_idx
                   + (jnp.arange(NUM_SEQ, dtype=jnp.int32) * VOCAB)[:, None, None]
                   ).reshape(-1)
    cat_tab = cat_tables.reshape(NUM_CAT * VOCAB, EMB)
    seq_tab = seq_tables.reshape(NUM_SEQ * VOCAB, EMB)
    cat_rows, seq = _make_sc_gather()(cat_idx_off, seq_idx_off, cat_tab, seq_tab)
    cat = cat_rows.reshape(B, CAT_COLS)
    return _mlp(dense, cat, seq, W1, b1, W2, b2, Wf, bf)
